# baseline (speedup 1.0000x reference)
import math

import jax
import jax.numpy as jnp
from jax import lax
from jax.experimental import pallas as pl

_B, _N, _FEAT = 64, 36, 2048
_K = 16
_NK = 8
_MID = 1024
_BM = 16
_BMN = _BM * _N


def _graph_conv_body(c2_ref, adj_ref, ti_ref, v_ref, wcat_ref,
                     mr_ref, mt_ref, pr_ref, pt_ref, out_ref):
    n, nk, k = _N, _NK, _K

    wv = lax.dot_general(v_ref[...], wcat_ref[...],
                         dimension_numbers=(((1,), (1,)), ((), ())),
                         preferred_element_type=jnp.float32)

    r_iota = lax.broadcasted_iota(jnp.int32, (2 * n, n), 0)
    c_iota = lax.broadcasted_iota(jnp.int32, (2 * n, n), 1)
    s_rho = (r_iota == 2 * c_iota).astype(jnp.float32)
    s_theta = (r_iota == 2 * c_iota + 1).astype(jnp.float32)
    c2 = c2_ref[...]
    rho = jnp.dot(c2, s_rho, preferred_element_type=jnp.float32)
    theta = jnp.dot(c2, s_theta, preferred_element_type=jnp.float32)

    lane = lax.broadcasted_iota(jnp.int32, (_BMN, n), 1)
    acc = jnp.zeros((_BMN, n), dtype=jnp.float32)
    for kk in range(k):
        acc = acc + jnp.where(lane == ti_ref[:, kk:kk + 1],
                              adj_ref[:, kk:kk + 1], 0.0)

    mr = mr_ref[...]
    mt = mt_ref[...]
    pr = pr_ref[...]
    pt = pt_ref[...]
    ws = []
    for i in range(nk):
        d = (rho - mr[0, i]) ** 2
        w_r = jnp.exp(-0.5 * d / (1e-14 + pr[0, i] ** 2))
        fa = jnp.abs(theta - mt[0, i])
        sa = jnp.abs(2.0 * math.pi - fa)
        ang = jnp.minimum(fa, sa)
        w_t = jnp.exp(-0.5 * ang * ang / (1e-14 + pt[0, i] ** 2))
        w = w_r * w_t
        w = jnp.where(jnp.isnan(w), 0.0, w)
        ws.append(w)
    wsum = ws[0]
    for i in range(1, nk):
        wsum = wsum + ws[i]
    scaled = acc / (wsum + 1e-14)

    for b in range(_BM):
        base = b * n
        wv_b = wv[base:base + n, :]
        sc_b = scaled[base:base + n, :]
        for i in range(nk):
            mi = ws[i][base:base + n, :] * sc_b
            out_ref[base:base + n, i * 128:(i + 1) * 128] = jnp.dot(
                mi, wv_b[:, i * 128:(i + 1) * 128],
                preferred_element_type=jnp.float32)


@jax.jit
def _graph_conv(v, c2, adj, ti, wcat, mr, mt, pr, pt):
    grid = _B // _BM
    out = pl.pallas_call(
        _graph_conv_body,
        grid=(grid,),
        in_specs=[
            pl.BlockSpec((_BMN, 2 * _N), lambda i: (i, 0)),
            pl.BlockSpec((_BMN, _K), lambda i: (i, 0)),
            pl.BlockSpec((_BMN, _K), lambda i: (i, 0)),
            pl.BlockSpec((_BMN, _FEAT), lambda i: (i, 0)),
            pl.BlockSpec((_MID, _FEAT), lambda i: (0, 0)),
            pl.BlockSpec((1, _NK), lambda i: (0, 0)),
            pl.BlockSpec((1, _NK), lambda i: (0, 0)),
            pl.BlockSpec((1, _NK), lambda i: (0, 0)),
            pl.BlockSpec((1, _NK), lambda i: (0, 0)),
        ],
        out_specs=pl.BlockSpec((_BMN, _MID), lambda i: (i, 0)),
        out_shape=jax.ShapeDtypeStruct((_B * _N, _MID), jnp.float32),
    )(c2, adj, ti, v, wcat, mr, mt, pr, pt)
    return out.reshape(_B, _N, _MID)


def kernel(v, v_mask, coord, adj_matrix, top_ind, W, mean_rho, mean_theta,
           precision_rho, precision_theta):
    del v_mask
    c2 = coord.reshape(_B * _N, 2 * _N)
    v2 = v.reshape(_B * _N, _FEAT)
    wcat = W.reshape(_MID, _FEAT)
    adj2 = adj_matrix.reshape(_B * _N, _K)
    ti2 = top_ind.reshape(_B * _N, _K).astype(jnp.int32)
    mr = mean_rho.reshape(1, _NK)
    mt = mean_theta.reshape(1, _NK)
    pr = precision_rho.reshape(1, _NK)
    pt = precision_theta.reshape(1, _NK)
    return _graph_conv(v2, c2, adj2, ti2, wcat, mr, mt, pr, pt)
